# baseline (speedup 1.0000x reference)
import functools

import jax
import jax.numpy as jnp
from jax import lax
from jax.experimental import pallas as pl
from jax.experimental.pallas import tpu as pltpu
from jax.experimental.pallas import tpu_sc as plsc

D_MODEL = 768
N_EXPERTS = 64
BT = 4096

NC = 2
NS = 16
NW = NC * NS
TPW = 1024
CHUNKS_PER_BATCH = 8


def _expert_mass(e1, e2, s1, s2):
    erow = jax.lax.broadcasted_iota(
        jnp.int32, (N_EXPERTS, e1.shape[1]), 0).astype(jnp.float32)
    m = jnp.where(erow == e1, s1, 0.0) + jnp.where(erow == e2, s2, 0.0)
    return jnp.sum(m, axis=1, keepdims=True)


def _router_body(x_ref, w_ref, scores_ref, idx_ref, loss_ref, counts_ref):
    i = pl.program_id(0)
    nblk = pl.num_programs(0)

    x = x_ref[...]
    w = w_ref[...]
    lt = jax.lax.dot_general(w, x, (((1,), (1,)), ((), ())),
                             preferred_element_type=jnp.float32)

    row = jax.lax.broadcasted_iota(jnp.int32, lt.shape, 0).astype(jnp.float32)
    m1 = jnp.max(lt, axis=0, keepdims=True)
    e1 = jnp.min(jnp.where(lt == m1, row, float(N_EXPERTS)),
                 axis=0, keepdims=True)
    lt2 = jnp.where(row == e1, -jnp.inf, lt)
    m2 = jnp.max(lt2, axis=0, keepdims=True)
    e2 = jnp.min(jnp.where(lt2 == m2, row, float(N_EXPERTS)),
                 axis=0, keepdims=True)

    ed = jnp.exp(m2 - m1)
    s1 = 1.0 / (1.0 + ed)
    s2 = ed * s1

    scores_ref[...] = jnp.concatenate([s1, s2], axis=0)[None]
    idx_t = jnp.concatenate([e1, e2], axis=0).astype(jnp.int32)
    idx_ref[...] = idx_t[None]

    @pl.when(i == 0)
    def _init():
        counts_ref[...] = jnp.zeros_like(counts_ref)

    counts_ref[...] += _expert_mass(e1, e2, s1, s2)

    @pl.when(i == nblk - 1)
    def _finish():
        counts = counts_ref[...]
        total = jnp.sum(counts)
        dev = counts / total * N_EXPERTS - 1.0
        loss_ref[...] = jnp.mean(dev * dev, axis=0, keepdims=True)


def _sc_gates_body(scores_hbm, idx_hbm, gates_hbm, sval_v, eidx_v, slab_v,
                   sem):
    wid = lax.axis_index("s") * NC + lax.axis_index("c")
    b = wid // CHUNKS_PER_BATCH
    chunk = wid % CHUNKS_PER_BATCH
    s0 = chunk * TPW

    for j in range(2):
        pltpu.sync_copy(scores_hbm.at[b, j, pl.ds(s0, TPW)],
                        sval_v.at[pl.ds(j * TPW, TPW)])
        pltpu.sync_copy(idx_hbm.at[b, j, pl.ds(s0, TPW)],
                        eidx_v.at[pl.ds(j * TPW, TPW)])

    zeros16 = jnp.zeros((16,), jnp.float32)
    for c in range(TPW // 16):
        slab_v[pl.ds(c * 16, 16)] = zeros16

    copies = []
    for e in range(N_EXPERTS):
        dst0 = (b * N_EXPERTS + e) * (CHUNKS_PER_BATCH * TPW) + s0
        copies.append(pltpu.async_copy(
            slab_v, gates_hbm.at[pl.ds(dst0, TPW)], sem))
    for cp in copies:
        cp.wait()

    tok_iota = lax.iota(jnp.int32, 16)
    base = (b * N_EXPERTS) * (CHUNKS_PER_BATCH * TPW) + s0
    scat = []
    for j in range(2):
        for k in range(TPW // 16):
            e16 = eidx_v[pl.ds(j * TPW + k * 16, 16)]
            f16 = base + e16 * (CHUNKS_PER_BATCH * TPW) + (tok_iota + k * 16)
            scat.append(pltpu.async_copy(
                sval_v.at[pl.ds(j * TPW + k * 16, 16)],
                gates_hbm.at[f16], sem))
    for cp in scat:
        cp.wait()


@functools.partial(jax.jit, static_argnums=())
def kernel(x, W, n_active, capacity):
    b, s, d = x.shape
    t = b * s
    blk_per_batch = s // BT
    xf = x.reshape(t, d)
    grid = (t // BT,)
    scores3, idx3, loss2d = pl.pallas_call(
        _router_body,
        grid=grid,
        in_specs=[
            pl.BlockSpec((BT, D_MODEL), lambda i: (i, 0)),
            pl.BlockSpec((N_EXPERTS, D_MODEL), lambda i: (0, 0)),
        ],
        out_specs=[
            pl.BlockSpec((1, 2, BT),
                         lambda i: (i // blk_per_batch, 0, i % blk_per_batch)),
            pl.BlockSpec((1, 2, BT),
                         lambda i: (i // blk_per_batch, 0, i % blk_per_batch)),
            pl.BlockSpec((1, 1), lambda i: (0, 0)),
        ],
        out_shape=[
            jax.ShapeDtypeStruct((b, 2, s), jnp.float32),
            jax.ShapeDtypeStruct((b, 2, s), jnp.int32),
            jax.ShapeDtypeStruct((1, 1), jnp.float32),
        ],
        scratch_shapes=[pltpu.VMEM((N_EXPERTS, 1), jnp.float32)],
    )(xf, W)

    mesh = plsc.VectorSubcoreMesh(core_axis_name="c", subcore_axis_name="s")
    gates_flat = pl.kernel(
        _sc_gates_body,
        mesh=mesh,
        out_type=jax.ShapeDtypeStruct((b * N_EXPERTS * s,), jnp.float32),
        scratch_types=[
            pltpu.VMEM((2 * TPW,), jnp.float32),
            pltpu.VMEM((2 * TPW,), jnp.int32),
            pltpu.VMEM((TPW,), jnp.float32),
            pltpu.SemaphoreType.DMA,
        ],
    )(scores3, idx3)

    gates = jnp.transpose(gates_flat.reshape(b, N_EXPERTS, s), (0, 2, 1))
    idx = jnp.transpose(idx3, (0, 2, 1))
    return gates, idx, loss2d[0, 0]


# speedup vs baseline: 3.7461x; 3.7461x over previous
import functools

import jax
import jax.numpy as jnp
from jax.experimental import pallas as pl
from jax.experimental.pallas import tpu as pltpu

D_MODEL = 768
N_EXPERTS = 64
BT = 4096


def _router_body(x_ref, w_ref, gates_ref, idx_ref, loss_ref, counts_ref):
    i = pl.program_id(0)
    nblk = pl.num_programs(0)

    x = x_ref[...]
    w = w_ref[...]
    lt = jax.lax.dot_general(w, x, (((1,), (1,)), ((), ())),
                             preferred_element_type=jnp.float32)

    row = jax.lax.broadcasted_iota(jnp.int32, lt.shape, 0).astype(jnp.float32)
    m1 = jnp.max(lt, axis=0, keepdims=True)
    e1 = jnp.min(jnp.where(lt == m1, row, float(N_EXPERTS)),
                 axis=0, keepdims=True)
    lt2 = jnp.where(row == e1, -jnp.inf, lt)
    m2 = jnp.max(lt2, axis=0, keepdims=True)
    e2 = jnp.min(jnp.where(lt2 == m2, row, float(N_EXPERTS)),
                 axis=0, keepdims=True)

    ed = jnp.exp(m2 - m1)
    s1 = 1.0 / (1.0 + ed)
    s2 = ed * s1

    gates_t = (jnp.where(row == e1, s1, 0.0)
               + jnp.where(row == e2, s2, 0.0))
    gates_ref[...] = gates_t[None]

    idx_t = jnp.concatenate([e1, e2], axis=0).astype(jnp.int32)
    idx_ref[...] = idx_t[None]

    @pl.when(i == 0)
    def _init():
        counts_ref[...] = jnp.zeros_like(counts_ref)

    counts_ref[...] += jnp.sum(gates_t, axis=1, keepdims=True)

    @pl.when(i == nblk - 1)
    def _finish():
        counts = counts_ref[...]
        total = jnp.sum(counts)
        dev = counts / total * N_EXPERTS - 1.0
        loss_ref[...] = jnp.mean(dev * dev, axis=0, keepdims=True)


@functools.partial(jax.jit, static_argnums=())
def kernel(x, W, n_active, capacity):
    b, s, d = x.shape
    t = b * s
    blk_per_batch = s // BT
    xf = x.reshape(t, d)
    grid = (t // BT,)
    gates3, idx3, loss2d = pl.pallas_call(
        _router_body,
        grid=grid,
        in_specs=[
            pl.BlockSpec((BT, D_MODEL), lambda i: (i, 0)),
            pl.BlockSpec((N_EXPERTS, D_MODEL), lambda i: (0, 0)),
        ],
        out_specs=[
            pl.BlockSpec((1, N_EXPERTS, BT),
                         lambda i: (i // blk_per_batch, 0, i % blk_per_batch)),
            pl.BlockSpec((1, 2, BT),
                         lambda i: (i // blk_per_batch, 0, i % blk_per_batch)),
            pl.BlockSpec((1, 1), lambda i: (0, 0)),
        ],
        out_shape=[
            jax.ShapeDtypeStruct((b, N_EXPERTS, s), jnp.float32),
            jax.ShapeDtypeStruct((b, 2, s), jnp.int32),
            jax.ShapeDtypeStruct((1, 1), jnp.float32),
        ],
        scratch_shapes=[pltpu.VMEM((N_EXPERTS, 1), jnp.float32)],
    )(xf, W)
    gates = jnp.transpose(gates3, (0, 2, 1))
    idx = jnp.transpose(idx3, (0, 2, 1))
    return gates, idx, loss2d[0, 0]
